# baseline (speedup 1.0000x reference)
import functools

import jax
import jax.numpy as jnp
from jax import lax
from jax.experimental import pallas as pl
from jax.experimental.pallas import tpu as pltpu
from jax.experimental.pallas import tpu_sc as plsc

N = 10000
D = 128
EPS = 1e-5

NC = 2
NS = 16
LANES = 128
NW = NC * NS
ACC_ROWS = 10112
ROWS_PER_TILE = ACC_ROWS // NS
DEG_ROWS = 10240
DEG_ROWS_PER_TILE = DEG_ROWS // NS

_mesh = plsc.VectorSubcoreMesh(core_axis_name="c", subcore_axis_name="s")
_mesh1 = plsc.VectorSubcoreMesh(
    core_axis_name="c", subcore_axis_name="s", num_cores=1)


def _sc_deg(dst2d, ones_l, zeros_1d):
    chunks = dst2d.shape[0] // NW

    @functools.partial(
        pl.kernel,
        out_type=jax.ShapeDtypeStruct((NC, DEG_ROWS), jnp.float32),
        mesh=_mesh,
        scratch_types=[
            pltpu.VMEM((chunks, LANES), jnp.int32),
            pltpu.VMEM((LANES,), jnp.float32),
            pltpu.VMEM_SHARED((DEG_ROWS,), jnp.float32),
        ],
    )
    def k(dst_hbm, ones_hbm, zeros_hbm, out_hbm, dst_v, ones_v, acc):
        c = lax.axis_index("c")
        s = lax.axis_index("s")
        rowbase = (c * NS + s) * chunks
        pltpu.sync_copy(dst_hbm.at[pl.ds(rowbase, chunks)], dst_v)
        pltpu.sync_copy(ones_hbm, ones_v)
        pltpu.sync_copy(
            zeros_hbm, acc.at[pl.ds(s * DEG_ROWS_PER_TILE, DEG_ROWS_PER_TILE)]
        )
        plsc.subcore_barrier()

        def body(j, carry):
            pltpu.sync_copy(ones_v, acc.at[dst_v.at[j]], add=True)
            return carry

        lax.fori_loop(0, chunks, body, 0)
        plsc.subcore_barrier()
        pltpu.sync_copy(
            acc.at[pl.ds(s * DEG_ROWS_PER_TILE, DEG_ROWS_PER_TILE)],
            out_hbm.at[c].at[pl.ds(s * DEG_ROWS_PER_TILE, DEG_ROWS_PER_TILE)],
        )

    return k(dst2d, ones_l, zeros_1d)


def _sc_scatter(y, src2d, dst2d, zeros_rows):
    total = src2d.shape[0]
    per_tile = total // NS
    segc = 32
    if per_tile % segc:
        segc = 16 if per_tile % 16 == 0 else 8
    nseg = per_tile // segc

    @functools.partial(
        pl.kernel,
        out_type=jax.ShapeDtypeStruct((ACC_ROWS, D), jnp.float32),
        mesh=_mesh1,
        scratch_types=[
            pltpu.VMEM((segc, LANES), jnp.int32),
            pltpu.VMEM((segc, LANES), jnp.int32),
            pltpu.VMEM((segc, LANES), jnp.int32),
            pltpu.VMEM((segc, LANES), jnp.int32),
            pltpu.VMEM((LANES, D), jnp.float32),
            pltpu.VMEM((LANES, D), jnp.float32),
            pltpu.VMEM_SHARED((ACC_ROWS, D), jnp.float32),
            pltpu.SemaphoreType.DMA,
            pltpu.SemaphoreType.DMA,
            pltpu.SemaphoreType.DMA,
        ],
    )
    def k(y_hbm, src_hbm, dst_hbm, zeros_hbm, out_hbm,
          src_sa, src_sb, dst_sa, dst_sb, rows_a, rows_b, acc,
          sem_a, sem_b, sem_i):
        s = lax.axis_index("s")
        rowbase = s * per_tile

        pltpu.async_copy(src_hbm.at[pl.ds(rowbase, segc)], src_sa, sem_i)
        pltpu.async_copy(dst_hbm.at[pl.ds(rowbase, segc)], dst_sa, sem_i)
        pltpu.sync_copy(zeros_hbm, acc.at[pl.ds(s * ROWS_PER_TILE, ROWS_PER_TILE)])
        plsc.subcore_barrier()

        bufs = [(src_sa, dst_sa), (src_sb, dst_sb)]
        for g in range(nseg):
            sseg, dseg = bufs[g % 2]
            snxt, dnxt = bufs[(g + 1) % 2]
            base = rowbase + g * segc
            pltpu.make_async_copy(src_hbm.at[pl.ds(base, segc)], sseg, sem_i).wait()
            pltpu.make_async_copy(dst_hbm.at[pl.ds(base, segc)], dseg, sem_i).wait()

            if g + 1 < nseg:
                nbase = rowbase + (g + 1) * segc
                pltpu.async_copy(src_hbm.at[pl.ds(nbase, segc)], snxt, sem_i)
                pltpu.async_copy(dst_hbm.at[pl.ds(nbase, segc)], dnxt, sem_i)

            pltpu.async_copy(y_hbm.at[sseg.at[0]], rows_a, sem_a)
            pltpu.async_copy(y_hbm.at[sseg.at[1]], rows_b, sem_b)

            def body(jj, carry, sseg=sseg, dseg=dseg):
                l0 = jj * 2
                l1 = l0 + 1
                pltpu.make_async_copy(y_hbm.at[sseg.at[l0]], rows_a, sem_a).wait()
                pltpu.sync_copy(rows_a, acc.at[dseg.at[l0]], add=True)

                @pl.when(l0 + 2 < segc)
                def _():
                    pltpu.async_copy(y_hbm.at[sseg.at[l0 + 2]], rows_a, sem_a)

                pltpu.make_async_copy(y_hbm.at[sseg.at[l1]], rows_b, sem_b).wait()
                pltpu.sync_copy(rows_b, acc.at[dseg.at[l1]], add=True)

                @pl.when(l1 + 2 < segc)
                def _():
                    pltpu.async_copy(y_hbm.at[sseg.at[l1 + 2]], rows_b, sem_b)

                return carry

            lax.fori_loop(0, segc // 2, body, 0)

        plsc.subcore_barrier()
        pltpu.sync_copy(
            acc.at[pl.ds(s * ROWS_PER_TILE, ROWS_PER_TILE)],
            out_hbm.at[pl.ds(s * ROWS_PER_TILE, ROWS_PER_TILE)],
        )

    return k(y, src2d, dst2d, zeros_rows)


def _tc_prep(x, W1, degp):
    def body(x_ref, w_ref, degp_ref, dinv_ref, y_ref):
        degp = degp_ref[...]
        dinv = lax.rsqrt(degp[0] + degp[1] + 1.0)[:ACC_ROWS]
        dinv_ref[...] = dinv
        xw = jnp.dot(x_ref[...], w_ref[...], preferred_element_type=jnp.float32)
        y_ref[...] = xw * dinv[:N]

    return pl.pallas_call(
        body,
        out_shape=[
            jax.ShapeDtypeStruct((ACC_ROWS, 1), jnp.float32),
            jax.ShapeDtypeStruct((N, D), jnp.float32),
        ],
    )(x, W1, degp)


def _tc_block(p, y, dinv, b, identity, Wn, g=None, be=None, bn=True):
    have_w = Wn is not None

    def body(*refs):
        if bn:
            p_ref, y_ref, dinv_ref, b_ref, id_ref, g_ref, be_ref = refs[:7]
            rest = refs[7:]
        else:
            p_ref, y_ref, dinv_ref, b_ref, id_ref = refs[:5]
            rest = refs[5:]
        if have_w:
            w_ref = rest[0]
            h_ref, yn_ref = rest[1], rest[2]
        else:
            h_ref = rest[0]
        dinv = dinv_ref[...][:N]
        pv = p_ref[...]
        t = (pv[:N] + y_ref[...]) * dinv + b_ref[...]
        if bn:
            mean = jnp.mean(t, axis=0, keepdims=True)
            var = jnp.mean((t - mean) ** 2, axis=0, keepdims=True)
            t = (t - mean) * lax.rsqrt(var + EPS) * g_ref[...] + be_ref[...]
            t = jnp.maximum(t, 0.0)
        h = t + id_ref[...]
        h_ref[...] = h
        if have_w:
            yn_ref[...] = (
                jnp.dot(h, w_ref[...], preferred_element_type=jnp.float32) * dinv
            )

    out_shape = [jax.ShapeDtypeStruct((N, D), jnp.float32)]
    if have_w:
        out_shape.append(jax.ShapeDtypeStruct((N, D), jnp.float32))
    args = [p, y, dinv, b.reshape(1, D), identity]
    if bn:
        args += [g.reshape(1, D), be.reshape(1, D)]
    if have_w:
        args.append(Wn)
    res = pl.pallas_call(body, out_shape=out_shape)(*args)
    return res if have_w else (res[0], None)


def kernel(x, edge_index, W1, b1, W2, b2, Wm, bm, W3, b3, W4, b4,
           g1, be1, g2, be2, g3, be3, g4, be4):
    src = edge_index[0]
    dst = edge_index[1]
    e = src.shape[0]
    block = NW * LANES * 8
    ep = ((e + block - 1) // block) * block
    pad = ep - e
    junk = N + (jnp.arange(pad, dtype=jnp.int32) % (ACC_ROWS - N))
    src_p = jnp.concatenate([src, jnp.zeros((pad,), jnp.int32)]).reshape(-1, LANES)
    dst_p = jnp.concatenate([dst, junk]).reshape(-1, LANES)

    ones_l = jnp.ones((LANES,), jnp.float32)
    zeros_1d = jnp.zeros((DEG_ROWS_PER_TILE,), jnp.float32)
    zeros_rows = jnp.zeros((ROWS_PER_TILE, D), jnp.float32)

    def scatter(y):
        return _sc_scatter(y, src_p, dst_p, zeros_rows)

    degp = _sc_deg(dst_p, ones_l, zeros_1d)
    dinv, y = _tc_prep(x, W1, degp.reshape(NC, DEG_ROWS, 1))

    h1, y = _tc_block(scatter(y), y, dinv, b1, x, W2, g1, be1, bn=True)
    h2, y = _tc_block(scatter(y), y, dinv, b2, h1, Wm, g2, be2, bn=True)
    dec, y = _tc_block(scatter(y), y, dinv, bm, h1, W3, bn=False)
    h3, y = _tc_block(scatter(y), y, dinv, b3, dec, W4, g3, be3, bn=True)
    out, _ = _tc_block(scatter(y), y, dinv, b4, h3, None, g4, be4, bn=True)
    return out


# speedup vs baseline: 2.5568x; 2.5568x over previous
import functools

import jax
import jax.numpy as jnp
from jax import lax
from jax.experimental import pallas as pl
from jax.experimental.pallas import tpu as pltpu
from jax.experimental.pallas import tpu_sc as plsc

N = 10000
D = 128
EPS = 1e-5

NC = 2
NS = 16
LANES = 128
NW = NC * NS
ACC_ROWS = 10112
ROWS_PER_TILE = ACC_ROWS // NS
DEG_ROWS = 10240
DEG_ROWS_PER_TILE = DEG_ROWS // NS

_mesh = plsc.VectorSubcoreMesh(core_axis_name="c", subcore_axis_name="s")
_mesh1 = plsc.VectorSubcoreMesh(
    core_axis_name="c", subcore_axis_name="s", num_cores=1)


def _sc_deg(dst2d, ones_l, zeros_1d):
    chunks = dst2d.shape[0] // NW

    @functools.partial(
        pl.kernel,
        out_type=jax.ShapeDtypeStruct((NC, DEG_ROWS), jnp.float32),
        mesh=_mesh,
        scratch_types=[
            pltpu.VMEM((chunks, LANES), jnp.int32),
            pltpu.VMEM((LANES,), jnp.float32),
            pltpu.VMEM_SHARED((DEG_ROWS,), jnp.float32),
        ],
    )
    def k(dst_hbm, ones_hbm, zeros_hbm, out_hbm, dst_v, ones_v, acc):
        c = lax.axis_index("c")
        s = lax.axis_index("s")
        rowbase = (c * NS + s) * chunks
        pltpu.sync_copy(dst_hbm.at[pl.ds(rowbase, chunks)], dst_v)
        pltpu.sync_copy(ones_hbm, ones_v)
        pltpu.sync_copy(
            zeros_hbm, acc.at[pl.ds(s * DEG_ROWS_PER_TILE, DEG_ROWS_PER_TILE)]
        )
        plsc.subcore_barrier()

        def body(j, carry):
            pltpu.sync_copy(ones_v, acc.at[dst_v.at[j]], add=True)
            return carry

        lax.fori_loop(0, chunks, body, 0)
        plsc.subcore_barrier()
        pltpu.sync_copy(
            acc.at[pl.ds(s * DEG_ROWS_PER_TILE, DEG_ROWS_PER_TILE)],
            out_hbm.at[c].at[pl.ds(s * DEG_ROWS_PER_TILE, DEG_ROWS_PER_TILE)],
        )

    return k(dst2d, ones_l, zeros_1d)


def _sc_scatter(y, src2d, dst2d, zeros_rows):
    total = src2d.shape[0]
    per_tile = total // NS
    segc = 32
    if per_tile % segc:
        segc = 16 if per_tile % 16 == 0 else 8
    nseg = per_tile // segc

    @functools.partial(
        pl.kernel,
        out_type=jax.ShapeDtypeStruct((ACC_ROWS, D), jnp.float32),
        mesh=_mesh1,
        scratch_types=[
            pltpu.VMEM((segc, LANES), jnp.int32),
            pltpu.VMEM((segc, LANES), jnp.int32),
            pltpu.VMEM((segc, LANES), jnp.int32),
            pltpu.VMEM((segc, LANES), jnp.int32),
            pltpu.VMEM((LANES, D), jnp.float32),
            pltpu.VMEM((LANES, D), jnp.float32),
            pltpu.VMEM_SHARED((ACC_ROWS, D), jnp.float32),
            pltpu.SemaphoreType.DMA,
            pltpu.SemaphoreType.DMA,
            pltpu.SemaphoreType.DMA,
        ],
    )
    def k(y_hbm, src_hbm, dst_hbm, zeros_hbm, out_hbm,
          src_sa, src_sb, dst_sa, dst_sb, rows_a, rows_b, acc,
          sem_a, sem_b, sem_i):
        s = lax.axis_index("s")
        rowbase = s * per_tile

        pltpu.async_copy(src_hbm.at[pl.ds(rowbase, segc)], src_sa, sem_i)
        pltpu.async_copy(dst_hbm.at[pl.ds(rowbase, segc)], dst_sa, sem_i)
        pltpu.sync_copy(zeros_hbm, acc.at[pl.ds(s * ROWS_PER_TILE, ROWS_PER_TILE)])
        plsc.subcore_barrier()

        bufs = [(src_sa, dst_sa), (src_sb, dst_sb)]
        for g in range(nseg):
            sseg, dseg = bufs[g % 2]
            snxt, dnxt = bufs[(g + 1) % 2]
            base = rowbase + g * segc
            pltpu.make_async_copy(src_hbm.at[pl.ds(base, segc)], sseg, sem_i).wait()
            pltpu.make_async_copy(dst_hbm.at[pl.ds(base, segc)], dseg, sem_i).wait()

            if g + 1 < nseg:
                nbase = rowbase + (g + 1) * segc
                pltpu.async_copy(src_hbm.at[pl.ds(nbase, segc)], snxt, sem_i)
                pltpu.async_copy(dst_hbm.at[pl.ds(nbase, segc)], dnxt, sem_i)

            pltpu.async_copy(y_hbm.at[sseg.at[0]], rows_a, sem_a)
            pltpu.async_copy(y_hbm.at[sseg.at[1]], rows_b, sem_b)

            def body(jj, carry, sseg=sseg, dseg=dseg):
                l0 = jj * 2
                l1 = l0 + 1
                pltpu.make_async_copy(y_hbm.at[sseg.at[l0]], rows_a, sem_a).wait()
                pltpu.sync_copy(rows_a, acc.at[dseg.at[l0]], add=True)

                @pl.when(l0 + 2 < segc)
                def _():
                    pltpu.async_copy(y_hbm.at[sseg.at[l0 + 2]], rows_a, sem_a)

                pltpu.make_async_copy(y_hbm.at[sseg.at[l1]], rows_b, sem_b).wait()
                pltpu.sync_copy(rows_b, acc.at[dseg.at[l1]], add=True)

                @pl.when(l1 + 2 < segc)
                def _():
                    pltpu.async_copy(y_hbm.at[sseg.at[l1 + 2]], rows_b, sem_b)

                return carry

            lax.fori_loop(0, segc // 2, body, 0)

        plsc.subcore_barrier()
        pltpu.sync_copy(
            acc.at[pl.ds(s * ROWS_PER_TILE, ROWS_PER_TILE)],
            out_hbm.at[pl.ds(s * ROWS_PER_TILE, ROWS_PER_TILE)],
        )

    return k(y, src2d, dst2d, zeros_rows)


def _tc_prep(x, W1, degp):
    def body(x_ref, w_ref, degp_ref, dinv_ref, y_ref):
        degp = degp_ref[...]
        dinv = lax.rsqrt(degp[0] + degp[1] + 1.0)[:ACC_ROWS]
        dinv_ref[...] = dinv
        xw = jnp.dot(x_ref[...], w_ref[...], preferred_element_type=jnp.float32)
        y_ref[...] = xw * dinv[:N]

    return pl.pallas_call(
        body,
        out_shape=[
            jax.ShapeDtypeStruct((ACC_ROWS, 1), jnp.float32),
            jax.ShapeDtypeStruct((N, D), jnp.float32),
        ],
    )(x, W1, degp)


def _tc_block(p, y, dinv, b, identity, Wn, g=None, be=None, bn=True):
    have_w = Wn is not None

    def body(*refs):
        if bn:
            p_ref, y_ref, dinv_ref, b_ref, id_ref, g_ref, be_ref = refs[:7]
            rest = refs[7:]
        else:
            p_ref, y_ref, dinv_ref, b_ref, id_ref = refs[:5]
            rest = refs[5:]
        if have_w:
            w_ref = rest[0]
            h_ref, yn_ref = rest[1], rest[2]
        else:
            h_ref = rest[0]
        dinv = dinv_ref[...][:N]
        pv = p_ref[...]
        t = (pv[:N] + y_ref[...]) * dinv + b_ref[...]
        if bn:
            mean = jnp.mean(t, axis=0, keepdims=True)
            var = jnp.mean((t - mean) ** 2, axis=0, keepdims=True)
            t = (t - mean) * lax.rsqrt(var + EPS) * g_ref[...] + be_ref[...]
            t = jnp.maximum(t, 0.0)
        h = t + id_ref[...]
        h_ref[...] = h
        if have_w:
            yn_ref[...] = (
                jnp.dot(h, w_ref[...], preferred_element_type=jnp.float32) * dinv
            )

    out_shape = [jax.ShapeDtypeStruct((N, D), jnp.float32)]
    if have_w:
        out_shape.append(jax.ShapeDtypeStruct((N, D), jnp.float32))
    args = [p, y, dinv, b.reshape(1, D), identity]
    if bn:
        args += [g.reshape(1, D), be.reshape(1, D)]
    if have_w:
        args.append(Wn)
    res = pl.pallas_call(body, out_shape=out_shape)(*args)
    return res if have_w else (res[0], None)


def kernel(x, edge_index, W1, b1, W2, b2, Wm, bm, W3, b3, W4, b4,
           g1, be1, g2, be2, g3, be3, g4, be4):
    src = edge_index[0]
    dst = edge_index[1]
    e = src.shape[0]
    block = NW * LANES * 8
    ep = ((e + block - 1) // block) * block
    pad = ep - e
    junk = N + (jnp.arange(pad, dtype=jnp.int32) % (ACC_ROWS - N))
    pad_src = jnp.arange(pad, dtype=jnp.int32) % N
    src_p = jnp.concatenate([src, pad_src]).reshape(-1, LANES)
    dst_p = jnp.concatenate([dst, junk]).reshape(-1, LANES)

    ones_l = jnp.ones((LANES,), jnp.float32)
    zeros_1d = jnp.zeros((DEG_ROWS_PER_TILE,), jnp.float32)
    zeros_rows = jnp.zeros((ROWS_PER_TILE, D), jnp.float32)

    def scatter(y):
        return _sc_scatter(y, src_p, dst_p, zeros_rows)

    degp = _sc_deg(dst_p, ones_l, zeros_1d)
    dinv, y = _tc_prep(x, W1, degp.reshape(NC, DEG_ROWS, 1))

    h1, y = _tc_block(scatter(y), y, dinv, b1, x, W2, g1, be1, bn=True)
    h2, y = _tc_block(scatter(y), y, dinv, b2, h1, Wm, g2, be2, bn=True)
    dec, y = _tc_block(scatter(y), y, dinv, bm, h1, W3, bn=False)
    h3, y = _tc_block(scatter(y), y, dinv, b3, dec, W4, g3, be3, bn=True)
    out, _ = _tc_block(scatter(y), y, dinv, b4, h3, None, g4, be4, bn=True)
    return out
